# baseline (speedup 1.0000x reference)
import jax
import jax.numpy as jnp
from jax.experimental import pallas as pl
from jax.experimental.pallas import tpu as pltpu

_H = 90
_T = 240
_F = 180
_LIVE_GATES = (0, 2, 3)


def _cells_kernel(x_ref, we_ref, be_ref, wd_ref, bd_ref, h2_ref):
    g = jnp.dot(x_ref[...], we_ref[...], preferred_element_type=jnp.float32)
    t = jnp.tanh(g.astype(jnp.bfloat16) + be_ref[...])
    ti, tg, to = t[:, 0:128], t[:, 128:256], t[:, 256:384]
    c = (0.5 * ti + 0.5) * tg
    hh = (0.25 * to + 0.25) * jnp.tanh(c)
    h1 = 0.5 * jnp.tanh(hh) + 0.5
    g2 = jnp.dot(h1, wd_ref[...], preferred_element_type=jnp.float32)
    t2 = jnp.tanh(g2.astype(jnp.bfloat16) + bd_ref[...])
    ti2, tg2, to2 = t2[:, 0:128], t2[:, 128:256], t2[:, 256:384]
    c2 = (0.5 * ti2 + 0.5) * tg2
    h2 = (0.5 * to2 + 0.5) * jnp.tanh(c2)
    h2_ref[...] = h2[:, 0:_H]


def _proj_kernel(h_ref, w_ref, b_ref, o_ref):
    z = jnp.dot(h_ref[...], w_ref[...], preferred_element_type=jnp.float32)
    z = z + b_ref[...]
    for gidx in range(4):
        zg = z[:, 10 * gidx:10 * (gidx + 1)]
        m = jnp.max(zg, axis=1, keepdims=True)
        e = jnp.exp(zg - m)
        s = jnp.sum(e, axis=1, keepdims=True)
        o_ref[:, 10 * gidx:10 * (gidx + 1)] = e / s


def kernel(x, enc_W_ih, enc_b_ih, enc_b_hh, dec_W_ih, dec_b_ih, dec_b_hh,
           out_W, out_b):
    B = x.shape[0]
    n_rows = B * _T

    xe = jnp.transpose(x, (0, 3, 1, 2)).reshape(n_rows, _F).astype(jnp.bfloat16)

    we = jnp.zeros((_F, 384), jnp.float32)
    wd = jnp.zeros((128, 384), jnp.float32)
    be = jnp.zeros((1, 384), jnp.float32)
    bd = jnp.zeros((1, 384), jnp.float32)
    eb = enc_b_ih + enc_b_hh
    db = dec_b_ih + dec_b_hh
    for slot, gate in enumerate(_LIVE_GATES):
        lo = 128 * slot
        scale = 0.5 if gate in (0, 3) else 1.0
        we = we.at[:, lo:lo + _H].set(scale * enc_W_ih[_H * gate:_H * (gate + 1), :].T)
        wd = wd.at[:_H, lo:lo + _H].set(scale * dec_W_ih[_H * gate:_H * (gate + 1), :].T)
        be = be.at[0, lo:lo + _H].set(scale * eb[_H * gate:_H * (gate + 1)])
        bd = bd.at[0, lo:lo + _H].set(scale * db[_H * gate:_H * (gate + 1)])
    we = we.astype(jnp.bfloat16)
    wd = wd.astype(jnp.bfloat16)
    be = be.astype(jnp.bfloat16)
    bd = bd.astype(jnp.bfloat16)

    rows_blk = 3840 if n_rows % 3840 == 0 else n_rows
    h2 = pl.pallas_call(
        _cells_kernel,
        out_shape=jax.ShapeDtypeStruct((n_rows, _H), jnp.bfloat16),
        grid=(n_rows // rows_blk,),
        in_specs=[
            pl.BlockSpec((rows_blk, _F), lambda i: (i, 0)),
            pl.BlockSpec((_F, 384), lambda i: (0, 0)),
            pl.BlockSpec((1, 384), lambda i: (0, 0)),
            pl.BlockSpec((128, 384), lambda i: (0, 0)),
            pl.BlockSpec((1, 384), lambda i: (0, 0)),
        ],
        out_specs=pl.BlockSpec((rows_blk, _H), lambda i: (i, 0)),
        compiler_params=pltpu.CompilerParams(
            dimension_semantics=("parallel",),
        ),
        name="lstm_cells",
    )(xe, we, be, wd, bd)

    hv = h2.reshape(B, _T * _H)
    w2 = out_W.T.astype(jnp.bfloat16)
    b2 = out_b.reshape(1, 40)
    b_blk = 128 if B % 128 == 0 else B
    out = pl.pallas_call(
        _proj_kernel,
        out_shape=jax.ShapeDtypeStruct((B, 40), jnp.float32),
        grid=(B // b_blk,),
        in_specs=[
            pl.BlockSpec((b_blk, _T * _H), lambda i: (i, 0)),
            pl.BlockSpec((_T * _H, 40), lambda i: (0, 0)),
            pl.BlockSpec((1, 40), lambda i: (0, 0)),
        ],
        out_specs=pl.BlockSpec((b_blk, 40), lambda i: (i, 0)),
        compiler_params=pltpu.CompilerParams(
            dimension_semantics=("parallel",),
        ),
        name="proj_softmax",
    )(hv, w2, b2)
    return out.reshape(B, 4, 10)
